# baseline (speedup 1.0000x reference)
import jax
import jax.numpy as jnp
from jax import lax
from jax.experimental import pallas as pl
from jax.experimental.pallas import tpu as pltpu
from jax.experimental.pallas import tpu_sc as plsc

VOCAB = 1_000_000
D = 64
B = 16384
NNEG = 20

NC = 2
NS = 16
NW = NC * NS
BW = B // NW
G = 16
NG = BW // G
NEGW = BW * NNEG
NEG_G = G * NNEG


def _issue(cent, ctxt, cidx_v, xidx_v, nidx_v, cen_b, ctx_b, neg_b, sem, g):
    pltpu.async_copy(cent.at[cidx_v.at[pl.ds(g * G, G)]], cen_b, sem)
    pltpu.async_copy(ctxt.at[xidx_v.at[pl.ds(g * G, G)]], ctx_b, sem)
    base = g * NEG_G
    pltpu.async_copy(ctxt.at[nidx_v.at[pl.ds(base, 128)]],
                     neg_b.at[pl.ds(0, 128)], sem)
    pltpu.async_copy(ctxt.at[nidx_v.at[pl.ds(base + 128, 128)]],
                     neg_b.at[pl.ds(128, 128)], sem)
    pltpu.async_copy(ctxt.at[nidx_v.at[pl.ds(base + 256, 64)]],
                     neg_b.at[pl.ds(256, 64)], sem)


def _drain(cent, ctxt, cidx_v, xidx_v, nidx_v, cen_b, ctx_b, neg_b, sem, g):
    pltpu.make_async_copy(
        cent.at[cidx_v.at[pl.ds(g * G, G)]], cen_b, sem).wait()
    pltpu.make_async_copy(
        ctxt.at[xidx_v.at[pl.ds(g * G, G)]], ctx_b, sem).wait()
    base = g * NEG_G
    pltpu.make_async_copy(ctxt.at[nidx_v.at[pl.ds(base, 128)]],
                          neg_b.at[pl.ds(0, 128)], sem).wait()
    pltpu.make_async_copy(ctxt.at[nidx_v.at[pl.ds(base + 128, 128)]],
                          neg_b.at[pl.ds(128, 128)], sem).wait()
    pltpu.make_async_copy(ctxt.at[nidx_v.at[pl.ds(base + 256, 64)]],
                          neg_b.at[pl.ds(256, 64)], sem).wait()


def _compute_group(cen_b, ctx_b, neg_b, pos_s, neg_s, g):
    iota = lax.iota(jnp.int32, 16)
    rows20 = iota * NNEG
    zero = jnp.zeros((16,), jnp.float32)

    def dbody(d0, carry):
        pos_acc, naccs = carry
        for du in range(4):
            dvec = (d0 * 4 + du + iota) & (D - 1)
            c = plsc.load_gather(cen_b, [iota, dvec])
            x = plsc.load_gather(ctx_b, [iota, dvec])
            pos_acc = pos_acc + c * x
            new_naccs = []
            for jj in range(NNEG):
                nvd = plsc.load_gather(neg_b, [rows20 + jj, dvec])
                new_naccs.append(naccs[jj] + c * nvd)
            naccs = tuple(new_naccs)
        return pos_acc, naccs

    pos_acc, naccs = lax.fori_loop(
        0, D // 4, dbody, (zero, tuple([zero] * NNEG)))
    pos_s[pl.ds(g * G, G)] = pos_acc
    for jj in range(NNEG):
        plsc.store_scatter(neg_s, [g * NEG_G + rows20 + jj], naccs[jj])


def _sc_body(cent, ctxt, cidx, xidx, nidx, pos_out, neg_out,
             cidx_v, xidx_v, nidx_v,
             cen_b0, ctx_b0, neg_b0, cen_b1, ctx_b1, neg_b1,
             pos_s, neg_s, sem0, sem1):
    w = lax.axis_index("s") * NC + lax.axis_index("c")

    pltpu.sync_copy(cidx.at[pl.ds(w * BW, BW)], cidx_v)
    pltpu.sync_copy(xidx.at[pl.ds(w * BW, BW)], xidx_v)
    pltpu.sync_copy(nidx.at[pl.ds(w * NEGW, NEGW)], nidx_v)

    args = (cent, ctxt, cidx_v, xidx_v, nidx_v)
    _issue(*args, cen_b0, ctx_b0, neg_b0, sem0, 0)

    def hbody(h, carry):
        g0 = 2 * h
        g1 = 2 * h + 1
        _issue(*args, cen_b1, ctx_b1, neg_b1, sem1, g1)
        _drain(*args, cen_b0, ctx_b0, neg_b0, sem0, g0)
        _compute_group(cen_b0, ctx_b0, neg_b0, pos_s, neg_s, g0)

        @pl.when(h + 1 < NG // 2)
        def _():
            _issue(*args, cen_b0, ctx_b0, neg_b0, sem0, g0 + 2)

        _drain(*args, cen_b1, ctx_b1, neg_b1, sem1, g1)
        _compute_group(cen_b1, ctx_b1, neg_b1, pos_s, neg_s, g1)
        return carry
    lax.fori_loop(0, NG // 2, hbody, 0)

    pltpu.sync_copy(pos_s, pos_out.at[pl.ds(w * BW, BW)])
    pltpu.sync_copy(neg_s, neg_out.at[pl.ds(w * NEGW, NEGW)])


@jax.jit
def _sc_scores(cent, ctxt, cidx, xidx, nidx):
    mesh = plsc.VectorSubcoreMesh(core_axis_name="c", subcore_axis_name="s")
    return pl.kernel(
        _sc_body,
        out_type=(
            jax.ShapeDtypeStruct((B,), jnp.float32),
            jax.ShapeDtypeStruct((B * NNEG,), jnp.float32),
        ),
        mesh=mesh,
        scratch_types=[
            pltpu.VMEM((BW,), jnp.int32),
            pltpu.VMEM((BW,), jnp.int32),
            pltpu.VMEM((NEGW,), jnp.int32),
            pltpu.VMEM((G, D), jnp.float32),
            pltpu.VMEM((G, D), jnp.float32),
            pltpu.VMEM((NEG_G, D), jnp.float32),
            pltpu.VMEM((G, D), jnp.float32),
            pltpu.VMEM((G, D), jnp.float32),
            pltpu.VMEM((NEG_G, D), jnp.float32),
            pltpu.VMEM((BW,), jnp.float32),
            pltpu.VMEM((NEGW,), jnp.float32),
            pltpu.SemaphoreType.DMA,
            pltpu.SemaphoreType.DMA,
        ],
        compiler_params=pltpu.CompilerParams(
            use_tc_tiling_on_sc=False,
            needs_layout_passes=False,
        ),
    )(cent, ctxt, cidx, xidx, nidx)


def _log_sigmoid(x):
    return jnp.minimum(x, 0.0) - jnp.log(1.0 + jnp.exp(-jnp.abs(x)))


def _tc_loss_body(pos_ref, neg_ref, out_ref):
    s = (jnp.sum(_log_sigmoid(pos_ref[...]))
         + jnp.sum(_log_sigmoid(-neg_ref[...])))
    out_ref[0, 0] = -s / B


@jax.jit
def _tc_loss(pos, neg):
    out = pl.pallas_call(
        _tc_loss_body,
        out_specs=pl.BlockSpec(memory_space=pltpu.SMEM),
        out_shape=jax.ShapeDtypeStruct((1, 1), jnp.float32),
    )(pos.reshape(128, 128), neg.reshape(2560, 128))
    return out[0, 0]


def kernel(center_table, context_table, center_words, context_words,
           negative_words):
    cidx = center_words.astype(jnp.int32)
    xidx = context_words.astype(jnp.int32)
    nidx = negative_words.astype(jnp.int32).reshape(-1)
    pos, neg = _sc_scores(center_table, context_table, cidx, xidx, nidx)
    return _tc_loss(pos, neg)


# speedup vs baseline: 1.0588x; 1.0588x over previous
import jax
import jax.numpy as jnp
from jax import lax
from jax.experimental import pallas as pl
from jax.experimental.pallas import tpu as pltpu
from jax.experimental.pallas import tpu_sc as plsc

VOCAB = 1_000_000
D = 64
B = 16384
NNEG = 20

NC = 2
NS = 16
NW = NC * NS
BW = B // NW
G = 16
NG = BW // G
NEGW = BW * NNEG
NEG_G = G * NNEG
PD = 128


def _issue(cent, ctxt, cidx_v, xidx_v, nidx_v, cen_b, ctx_b, neg_b, sem, g):
    pltpu.async_copy(cent.at[cidx_v.at[pl.ds(g * G, G)]], cen_b, sem)
    pltpu.async_copy(ctxt.at[xidx_v.at[pl.ds(g * G, G)]], ctx_b, sem)
    base = g * NEG_G
    pltpu.async_copy(ctxt.at[nidx_v.at[pl.ds(base, 128)]],
                     neg_b.at[pl.ds(0, 128)], sem)
    pltpu.async_copy(ctxt.at[nidx_v.at[pl.ds(base + 128, 128)]],
                     neg_b.at[pl.ds(128, 128)], sem)
    pltpu.async_copy(ctxt.at[nidx_v.at[pl.ds(base + 256, 64)]],
                     neg_b.at[pl.ds(256, 64)], sem)


def _drain(cent, ctxt, cidx_v, xidx_v, nidx_v, cen_b, ctx_b, neg_b, sem, g):
    pltpu.make_async_copy(
        cent.at[cidx_v.at[pl.ds(g * G, G)]], cen_b, sem).wait()
    pltpu.make_async_copy(
        ctxt.at[xidx_v.at[pl.ds(g * G, G)]], ctx_b, sem).wait()
    base = g * NEG_G
    pltpu.make_async_copy(ctxt.at[nidx_v.at[pl.ds(base, 128)]],
                          neg_b.at[pl.ds(0, 128)], sem).wait()
    pltpu.make_async_copy(ctxt.at[nidx_v.at[pl.ds(base + 128, 128)]],
                          neg_b.at[pl.ds(128, 128)], sem).wait()
    pltpu.make_async_copy(ctxt.at[nidx_v.at[pl.ds(base + 256, 64)]],
                          neg_b.at[pl.ds(256, 64)], sem).wait()


def _compute_group(cen_b, ctx_b, neg_b, pos_s, neg_s, g):
    iota = lax.iota(jnp.int32, 16)
    rows20 = iota * NNEG
    zero = jnp.zeros((16,), jnp.float32)

    def dbody(d0, carry):
        pos_acc, naccs = carry
        for du in range(4):
            dvec = (d0 * 4 + du + iota) & (D - 1)
            c = plsc.load_gather(cen_b, [iota, dvec])
            x = plsc.load_gather(ctx_b, [iota, dvec])
            pos_acc = pos_acc + c * x
            new_naccs = []
            for jj in range(NNEG):
                nvd = plsc.load_gather(neg_b, [rows20 + jj, dvec])
                new_naccs.append(naccs[jj] + c * nvd)
            naccs = tuple(new_naccs)
        return pos_acc, naccs

    pos_acc, naccs = lax.fori_loop(
        0, D // 4, dbody, (zero, tuple([zero] * NNEG)))
    pos_s[pl.ds(g * G, G)] = pos_acc
    for jj in range(NNEG):
        plsc.store_scatter(neg_s, [g * NEG_G + rows20 + jj], naccs[jj])


def _sc_body(cent, ctxt, cidx, xidx, nidx, pos_out, neg_out,
             cidx_v, xidx_v, nidx_v,
             cen_b0, ctx_b0, neg_b0, cen_b1, ctx_b1, neg_b1,
             pos_s, neg_s, sem0, sem1):
    w = lax.axis_index("s") * NC + lax.axis_index("c")

    pltpu.sync_copy(cidx.at[pl.ds(w * BW, BW)], cidx_v)
    pltpu.sync_copy(xidx.at[pl.ds(w * BW, BW)], xidx_v)
    pltpu.sync_copy(nidx.at[pl.ds(w * NEGW, NEGW)], nidx_v)

    args = (cent, ctxt, cidx_v, xidx_v, nidx_v)
    _issue(*args, cen_b0, ctx_b0, neg_b0, sem0, 0)

    def hbody(h, carry):
        g0 = 2 * h
        g1 = 2 * h + 1
        _issue(*args, cen_b1, ctx_b1, neg_b1, sem1, g1)
        _drain(*args, cen_b0, ctx_b0, neg_b0, sem0, g0)
        _compute_group(cen_b0, ctx_b0, neg_b0, pos_s, neg_s, g0)

        @pl.when(h + 1 < NG // 2)
        def _():
            _issue(*args, cen_b0, ctx_b0, neg_b0, sem0, g0 + 2)

        _drain(*args, cen_b1, ctx_b1, neg_b1, sem1, g1)
        _compute_group(cen_b1, ctx_b1, neg_b1, pos_s, neg_s, g1)
        return carry
    lax.fori_loop(0, NG // 2, hbody, 0)

    pltpu.sync_copy(pos_s, pos_out.at[pl.ds(w * BW, BW)])
    pltpu.sync_copy(neg_s, neg_out.at[pl.ds(w * NEGW, NEGW)])


@jax.jit
def _sc_scores(cent, ctxt, cidx, xidx, nidx):
    mesh = plsc.VectorSubcoreMesh(core_axis_name="c", subcore_axis_name="s")
    return pl.kernel(
        _sc_body,
        out_type=(
            jax.ShapeDtypeStruct((B,), jnp.float32),
            jax.ShapeDtypeStruct((B * NNEG,), jnp.float32),
        ),
        mesh=mesh,
        scratch_types=[
            pltpu.VMEM((BW,), jnp.int32),
            pltpu.VMEM((BW,), jnp.int32),
            pltpu.VMEM((NEGW,), jnp.int32),
            pltpu.VMEM((G, PD), jnp.float32),
            pltpu.VMEM((G, PD), jnp.float32),
            pltpu.VMEM((NEG_G, PD), jnp.float32),
            pltpu.VMEM((G, PD), jnp.float32),
            pltpu.VMEM((G, PD), jnp.float32),
            pltpu.VMEM((NEG_G, PD), jnp.float32),
            pltpu.VMEM((BW,), jnp.float32),
            pltpu.VMEM((NEGW,), jnp.float32),
            pltpu.SemaphoreType.DMA,
            pltpu.SemaphoreType.DMA,
        ],
        compiler_params=pltpu.CompilerParams(
            needs_layout_passes=False,
        ),
    )(cent, ctxt, cidx, xidx, nidx)


def _log_sigmoid(x):
    return jnp.minimum(x, 0.0) - jnp.log(1.0 + jnp.exp(-jnp.abs(x)))


def _tc_loss_body(pos_ref, neg_ref, out_ref):
    s = (jnp.sum(_log_sigmoid(pos_ref[...]))
         + jnp.sum(_log_sigmoid(-neg_ref[...])))
    out_ref[0, 0] = -s / B


@jax.jit
def _tc_loss(pos, neg):
    out = pl.pallas_call(
        _tc_loss_body,
        out_specs=pl.BlockSpec(memory_space=pltpu.SMEM),
        out_shape=jax.ShapeDtypeStruct((1, 1), jnp.float32),
    )(pos.reshape(128, 128), neg.reshape(2560, 128))
    return out[0, 0]


def kernel(center_table, context_table, center_words, context_words,
           negative_words):
    cidx = center_words.astype(jnp.int32)
    xidx = context_words.astype(jnp.int32)
    nidx = negative_words.astype(jnp.int32).reshape(-1)
    cent = jnp.pad(center_table, ((0, 0), (0, PD - D)))
    ctxt = jnp.pad(context_table, ((0, 0), (0, PD - D)))
    pos, neg = _sc_scores(cent, ctxt, cidx, xidx, nidx)
    return _tc_loss(pos, neg)
